# Initial kernel scaffold; baseline (speedup 1.0000x reference)
import jax
import jax.numpy as jnp
from jax.experimental import pallas as pl


def kernel(x, edge_index, aug_type):
    raise NotImplementedError("write your pallas kernel here")


# speedup vs baseline: 1.7697x; 1.7697x over previous
import jax
import jax.numpy as jnp
from jax import lax
from jax.experimental import pallas as pl
from jax.experimental.pallas import tpu as pltpu
from jax.experimental.pallas import tpu_sc as plsc

_N = 10000
_E = 320000
_D = 128
_MASK_NUM = 1500
_REPL_NUM = 150
_NC = 2
_NS = 16
_NW = _NC * _NS
_EPW = 2 * _E // _NW
_RPW = 320
_LAST_BASE = _N - _RPW
_NPAD = _N + 16
_IMIN = jnp.iinfo(jnp.int32).min


def _sget(ref, i):
    return ref[pl.ds(i, 16)][0]


def _bincount_body(edges_hbm, hist_hbm, ev, hv):
    c = lax.axis_index("c")
    s = lax.axis_index("s")
    wid = s * _NC + c
    base = wid * _EPW
    pltpu.sync_copy(edges_hbm.at[pl.ds(base, _EPW)], ev)
    z = jnp.zeros((16,), jnp.int32)

    def zero_body(i, _):
        hv[pl.ds(i * 16, 16)] = z
        return 0

    lax.fori_loop(0, _N // 16, zero_body, 0, unroll=8)
    ones = jnp.ones((16,), jnp.int32)

    def scat_body(i, _):
        idx = ev[pl.ds(i * 16, 16)]
        plsc.addupdate_scatter(hv, [idx], ones)
        return 0

    lax.fori_loop(0, _EPW // 16, scat_body, 0, unroll=8)
    pltpu.sync_copy(hv, hist_hbm.at[pl.ds(wid * _N, _N)])


def _sc_bincount(edges):
    mesh = plsc.VectorSubcoreMesh(core_axis_name="c", subcore_axis_name="s")
    return pl.kernel(
        _bincount_body,
        out_type=jax.ShapeDtypeStruct((_NW * _N,), jnp.int32),
        mesh=mesh,
        scratch_types=[
            pltpu.VMEM((_EPW,), jnp.int32),
            pltpu.VMEM((_N,), jnp.int32),
        ],
        compiler_params=pltpu.CompilerParams(needs_layout_passes=False),
    )(edges)


def _tc_score_body(hist_ref, gum_ref, mask_ref, skey_ref, tz_ref, tr_ref):
    deg = jnp.sum(hist_ref[...], axis=0, keepdims=True).astype(jnp.float32)
    m = jnp.max(deg)
    e = jnp.exp(deg - m)
    ssum = jnp.sum(e)
    prob = e / ssum
    sc = jnp.log(prob + jnp.float32(1e-20)) + gum_ref[...]
    b = lax.bitcast_convert_type(sc, jnp.int32)
    u = jnp.where(
        b < 0,
        lax.bitcast_convert_type(~b, jnp.uint32),
        lax.bitcast_convert_type(b, jnp.uint32) | jnp.uint32(0x80000000),
    )

    def kth_largest(k):
        def bit_body(t, pfx):
            sh = jnp.uint32(31) - t.astype(jnp.uint32)
            cand = pfx | (jnp.uint32(1) << sh)
            cnt = jnp.sum((u >= cand).astype(jnp.int32))
            return jnp.where(cnt >= k, cand, pfx)

        return lax.fori_loop(0, 32, bit_body, jnp.uint32(0))

    tz = kth_largest(_MASK_NUM)
    tr = kth_largest(_REPL_NUM)
    flip = jnp.uint32(0x80000000)
    mask_ref[...] = (u >= tz).astype(jnp.int32)
    skey_ref[...] = lax.bitcast_convert_type(u ^ flip, jnp.int32)
    tz_ref[...] = jnp.full(
        (1, 16), lax.bitcast_convert_type(tz ^ flip, jnp.int32), jnp.int32)
    tr_ref[...] = jnp.full(
        (1, 16), lax.bitcast_convert_type(tr ^ flip, jnp.int32), jnp.int32)


def _tc_scores(hist, gum2):
    return pl.pallas_call(
        _tc_score_body,
        out_shape=[
            jax.ShapeDtypeStruct((1, _N), jnp.int32),
            jax.ShapeDtypeStruct((1, _N), jnp.int32),
            jax.ShapeDtypeStruct((1, 16), jnp.int32),
            jax.ShapeDtypeStruct((1, 16), jnp.int32),
        ],
    )(hist, gum2)


def _sc_apply_body(x_hbm, skey_hbm, tz_hbm, tr_hbm, rand_hbm, out_hbm,
                   buf, ub, tzv, trv, ckey, cidx, lidx, olist):
    c = lax.axis_index("c")
    s = lax.axis_index("s")
    wid = s * _NC + c
    base = jnp.minimum(wid * _RPW, _LAST_BASE)
    pltpu.sync_copy(x_hbm.at[pl.ds(base * _D, _RPW * _D)], buf)
    pltpu.sync_copy(skey_hbm, ub)
    pltpu.sync_copy(tz_hbm, tzv)
    pltpu.sync_copy(tr_hbm, trv)
    tz = tzv[...]
    tr = trv[...]
    lane = lax.iota(jnp.int32, 16)
    imin = jnp.full((16,), _IMIN, jnp.int32)
    for q in range(11):
        ckey[pl.ds(q * 16, 16)] = imin

    def zscan(i, cnt):
        v = ub[pl.ds(base + i * 16, 16)]
        msk = v >= tz
        lv = i * 16 + lane
        plsc.store_compressed(lidx.at[pl.ds(cnt, 16)], lv, mask=msk)
        return cnt + jnp.sum(msk.astype(jnp.int32))

    zcnt = lax.fori_loop(0, 20, zscan, jnp.int32(0))
    zrow = jnp.zeros((16,), jnp.float32)

    def zero_body(j, _):
        r = _sget(lidx, j)
        for cc in range(8):
            buf[pl.ds(r * _D + cc * 16, 16)] = zrow
        return 0

    lax.fori_loop(0, zcnt, zero_body, 0)

    def cscan(i, cnt):
        v = ub[pl.ds(i * 16, 16)]
        msk = v >= tr
        gv = i * 16 + lane
        plsc.store_compressed(ckey.at[pl.ds(cnt, 16)], v, mask=msk)
        plsc.store_compressed(cidx.at[pl.ds(cnt, 16)], gv, mask=msk)
        return cnt + jnp.sum(msk.astype(jnp.int32))

    ccnt = lax.fori_loop(0, _NPAD // 16, cscan, jnp.int32(0))

    def oscan(j, cnt):
        gi = _sget(cidx, j)
        own = (j < ccnt) & (gi >= base) & (gi < base + _RPW)
        olist[pl.ds(cnt, 16)] = jnp.full((16,), j, jnp.int32)
        return cnt + own.astype(jnp.int32)

    ocnt = lax.fori_loop(0, 160, oscan, jnp.int32(0))

    def repl_body(jj, _):
        j = _sget(olist, jj)
        kj = _sget(ckey, j)
        gi = _sget(cidx, j)
        kjv = jnp.full((16,), kj, jnp.int32)
        giv = jnp.full((16,), gi, jnp.int32)

        def rk(q, r):
            ck = ckey[pl.ds(q * 16, 16)]
            ci = cidx[pl.ds(q * 16, 16)]
            cmp = (ck > kjv) | ((ck == kjv) & (ci < giv))
            return r + jnp.sum(cmp.astype(jnp.int32))

        rank = lax.fori_loop(0, 11, rk, jnp.int32(0))
        rank = jnp.minimum(rank, _REPL_NUM - 1)
        pltpu.sync_copy(rand_hbm.at[pl.ds(rank * _D, _D)],
                        buf.at[pl.ds((gi - base) * _D, _D)])
        return 0

    lax.fori_loop(0, ocnt, repl_body, 0)
    pltpu.sync_copy(buf, out_hbm.at[pl.ds(base * _D, _RPW * _D)])


def _sc_apply(x, skey_pad, tzv, trv, rand):
    mesh = plsc.VectorSubcoreMesh(core_axis_name="c", subcore_axis_name="s")
    return pl.kernel(
        _sc_apply_body,
        out_type=jax.ShapeDtypeStruct((_N * _D,), jnp.float32),
        mesh=mesh,
        scratch_types=[
            pltpu.VMEM((_RPW * _D,), jnp.float32),
            pltpu.VMEM((_NPAD,), jnp.int32),
            pltpu.VMEM((16,), jnp.int32),
            pltpu.VMEM((16,), jnp.int32),
            pltpu.VMEM((176,), jnp.int32),
            pltpu.VMEM((176,), jnp.int32),
            pltpu.VMEM((352,), jnp.int32),
            pltpu.VMEM((192,), jnp.int32),
        ],
        compiler_params=pltpu.CompilerParams(needs_layout_passes=False),
    )(x, skey_pad, tzv, trv, rand)


def kernel(x, edge_index, aug_type):
    del aug_type
    key = jax.random.key(42)
    kg, kr = jax.random.split(key)
    gum = jax.random.gumbel(kg, (_N,), dtype=jnp.float32)
    rand = jax.random.normal(kr, (_REPL_NUM, _D), dtype=x.dtype)
    edges = edge_index.reshape(2 * _E)
    hist = _sc_bincount(edges).reshape(_NW, _N)
    mask_i, skey, tzv, trv = _tc_scores(hist, gum.reshape(1, _N))
    skey_pad = jnp.concatenate(
        [skey.reshape(_N), jnp.full((16,), _IMIN, jnp.int32)])
    out = _sc_apply(x.reshape(_N * _D), skey_pad,
                    tzv.reshape(16), trv.reshape(16), rand.reshape(-1))
    return out.reshape(_N, _D), mask_i.reshape(_N) != 0


# speedup vs baseline: 1.9105x; 1.0796x over previous
import jax
import jax.numpy as jnp
from jax import lax
from jax.experimental import pallas as pl
from jax.experimental.pallas import tpu as pltpu
from jax.experimental.pallas import tpu_sc as plsc

_N = 10000
_E = 320000
_D = 128
_MASK_NUM = 1500
_REPL_NUM = 150
_NC = 2
_NS = 16
_NW = _NC * _NS
_EPW = 2 * _E // _NW
_RPW = 320
_LAST_BASE = _N - _RPW
_NPAD = _N + 16
_IMIN = jnp.iinfo(jnp.int32).min

def _op_constants():
    kg, kr = jax.random.split(jax.random.key(42))
    gum_pad = jnp.pad(
        jax.random.gumbel(kg, (_N,), dtype=jnp.float32),
        (0, _NPAD - _N)).reshape(1, _NPAD)
    rand_flat = jax.random.normal(
        kr, (_REPL_NUM, _D), dtype=jnp.float32).reshape(-1)
    return gum_pad, rand_flat


def _sget(ref, i):
    return ref[pl.ds(i, 16)][0]


def _bincount_body(edges_hbm, hist_hbm, ev, hv):
    c = lax.axis_index("c")
    s = lax.axis_index("s")
    wid = s * _NC + c
    base = wid * _EPW
    pltpu.sync_copy(edges_hbm.at[pl.ds(base, _EPW)], ev)
    z = jnp.zeros((16,), jnp.int32)

    @plsc.parallel_loop(0, _NPAD // 16, unroll=8)
    def _(i):
        hv[pl.ds(i * 16, 16)] = z

    ones = jnp.ones((16,), jnp.int32)

    @plsc.parallel_loop(0, _EPW // 16, unroll=8)
    def _(i):
        idx = ev[pl.ds(i * 16, 16)]
        plsc.addupdate_scatter(hv, [idx], ones)

    pltpu.sync_copy(hv, hist_hbm.at[pl.ds(wid * _NPAD, _NPAD)])


def _sc_bincount(edges):
    mesh = plsc.VectorSubcoreMesh(core_axis_name="c", subcore_axis_name="s")
    return pl.kernel(
        _bincount_body,
        out_type=jax.ShapeDtypeStruct((_NW * _NPAD,), jnp.int32),
        mesh=mesh,
        scratch_types=[
            pltpu.VMEM((_EPW,), jnp.int32),
            pltpu.VMEM((_NPAD,), jnp.int32),
        ],
        compiler_params=pltpu.CompilerParams(needs_layout_passes=False),
    )(edges)


def _tc_score_body(hist_ref, gum_ref, mask_ref, skey_ref, tz_ref, tr_ref):
    col = lax.broadcasted_iota(jnp.int32, (1, _NPAD), 1)
    valid = col < _N
    deg = jnp.sum(hist_ref[...], axis=0, keepdims=True).astype(jnp.float32)
    m = jnp.max(deg)
    e = jnp.where(valid, jnp.exp(deg - m), jnp.float32(0.0))
    ssum = jnp.sum(e)
    prob = e / ssum
    sc = jnp.log(prob + jnp.float32(1e-20)) + gum_ref[...]
    b = lax.bitcast_convert_type(sc, jnp.int32)
    u = jnp.where(
        b < 0,
        lax.bitcast_convert_type(~b, jnp.uint32),
        lax.bitcast_convert_type(b, jnp.uint32) | jnp.uint32(0x80000000),
    )
    u = jnp.where(valid, u, jnp.uint32(0))

    def bit_body(t, ps):
        p1, p2 = ps
        sh = jnp.uint32(31) - t.astype(jnp.uint32)
        one = jnp.uint32(1) << sh
        c1 = p1 | one
        c2 = p2 | one
        n1 = jnp.sum((u >= c1).astype(jnp.int32))
        n2 = jnp.sum((u >= c2).astype(jnp.int32))
        return (jnp.where(n1 >= _MASK_NUM, c1, p1),
                jnp.where(n2 >= _REPL_NUM, c2, p2))

    tz, tr = lax.fori_loop(0, 32, bit_body, (jnp.uint32(0), jnp.uint32(0)))
    flip = jnp.uint32(0x80000000)
    mask_ref[...] = (u >= tz).astype(jnp.int32)[:, :_N]
    skey_ref[...] = lax.bitcast_convert_type(u ^ flip, jnp.int32)
    tz_ref[...] = jnp.full(
        (1, 16), lax.bitcast_convert_type(tz ^ flip, jnp.int32), jnp.int32)
    tr_ref[...] = jnp.full(
        (1, 16), lax.bitcast_convert_type(tr ^ flip, jnp.int32), jnp.int32)


def _tc_scores(hist, gum2):
    return pl.pallas_call(
        _tc_score_body,
        out_shape=[
            jax.ShapeDtypeStruct((1, _N), jnp.int32),
            jax.ShapeDtypeStruct((1, _NPAD), jnp.int32),
            jax.ShapeDtypeStruct((1, 16), jnp.int32),
            jax.ShapeDtypeStruct((1, 16), jnp.int32),
        ],
    )(hist, gum2)


def _sc_apply_body(x_hbm, skey_hbm, tz_hbm, tr_hbm, rand_hbm, out_hbm,
                   buf, ub, tzv, trv, ckey, cidx, lidx, olist):
    c = lax.axis_index("c")
    s = lax.axis_index("s")
    wid = s * _NC + c
    base = jnp.minimum(wid * _RPW, _LAST_BASE)
    pltpu.sync_copy(x_hbm.at[pl.ds(base * _D, _RPW * _D)], buf)
    pltpu.sync_copy(skey_hbm, ub)
    pltpu.sync_copy(tz_hbm, tzv)
    pltpu.sync_copy(tr_hbm, trv)
    tz = tzv[...]
    tr = trv[...]
    lane = lax.iota(jnp.int32, 16)
    imin = jnp.full((16,), _IMIN, jnp.int32)
    for q in range(11):
        ckey[pl.ds(q * 16, 16)] = imin

    def zscan(i, cnt):
        v = ub[pl.ds(base + i * 16, 16)]
        msk = v >= tz
        lv = i * 16 + lane
        plsc.store_compressed(lidx.at[pl.ds(cnt, 16)], lv, mask=msk)
        return cnt + plsc.all_reduce_population_count(msk)[0]

    zcnt = lax.fori_loop(0, 20, zscan, jnp.int32(0))
    zrow = jnp.zeros((16,), jnp.float32)

    def zero_body(j, _):
        r = _sget(lidx, j)
        for cc in range(8):
            buf[pl.ds(r * _D + cc * 16, 16)] = zrow
        return 0

    lax.fori_loop(0, zcnt, zero_body, 0)

    def cscan(i, cnt):
        v = ub[pl.ds(i * 16, 16)]
        msk = v >= tr
        gv = i * 16 + lane
        plsc.store_compressed(ckey.at[pl.ds(cnt, 16)], v, mask=msk)
        plsc.store_compressed(cidx.at[pl.ds(cnt, 16)], gv, mask=msk)
        return cnt + plsc.all_reduce_population_count(msk)[0]

    ccnt = lax.fori_loop(0, _NPAD // 16, cscan, jnp.int32(0))

    def oscan(j, cnt):
        gi = _sget(cidx, j)
        own = (j < ccnt) & (gi >= base) & (gi < base + _RPW)
        olist[pl.ds(cnt, 16)] = jnp.full((16,), j, jnp.int32)
        return cnt + own.astype(jnp.int32)

    ocnt = lax.fori_loop(0, 160, oscan, jnp.int32(0))

    def repl_body(jj, _):
        j = _sget(olist, jj)
        kj = _sget(ckey, j)
        gi = _sget(cidx, j)
        kjv = jnp.full((16,), kj, jnp.int32)
        giv = jnp.full((16,), gi, jnp.int32)

        def rk(q, r):
            ck = ckey[pl.ds(q * 16, 16)]
            ci = cidx[pl.ds(q * 16, 16)]
            cmp = (ck > kjv) | ((ck == kjv) & (ci < giv))
            return r + plsc.all_reduce_population_count(cmp)[0]

        rank = lax.fori_loop(0, 11, rk, jnp.int32(0))
        rank = jnp.minimum(rank, _REPL_NUM - 1)
        pltpu.sync_copy(rand_hbm.at[pl.ds(rank * _D, _D)],
                        buf.at[pl.ds((gi - base) * _D, _D)])
        return 0

    lax.fori_loop(0, ocnt, repl_body, 0)
    pltpu.sync_copy(buf, out_hbm.at[pl.ds(base * _D, _RPW * _D)])


def _sc_apply(x, skey_pad, tzv, trv, rand):
    mesh = plsc.VectorSubcoreMesh(core_axis_name="c", subcore_axis_name="s")
    return pl.kernel(
        _sc_apply_body,
        out_type=jax.ShapeDtypeStruct((_N * _D,), jnp.float32),
        mesh=mesh,
        scratch_types=[
            pltpu.VMEM((_RPW * _D,), jnp.float32),
            pltpu.VMEM((_NPAD,), jnp.int32),
            pltpu.VMEM((16,), jnp.int32),
            pltpu.VMEM((16,), jnp.int32),
            pltpu.VMEM((176,), jnp.int32),
            pltpu.VMEM((176,), jnp.int32),
            pltpu.VMEM((352,), jnp.int32),
            pltpu.VMEM((192,), jnp.int32),
        ],
        compiler_params=pltpu.CompilerParams(needs_layout_passes=False),
    )(x, skey_pad, tzv, trv, rand)


def kernel(x, edge_index, aug_type):
    del aug_type
    gum_pad, rand_flat = _op_constants()
    edges = edge_index.reshape(2 * _E)
    hist = _sc_bincount(edges).reshape(_NW, _NPAD)
    mask_i, skey, tzv, trv = _tc_scores(hist, gum_pad)
    out = _sc_apply(x.reshape(_N * _D), skey.reshape(_NPAD),
                    tzv.reshape(16), trv.reshape(16), rand_flat)
    return out.reshape(_N, _D), mask_i.reshape(_N) != 0
